# baseline (speedup 1.0000x reference)
import functools

import jax
import jax.numpy as jnp
from jax.experimental import pallas as pl
from jax.experimental.pallas import tpu as pltpu

_BN_EPS = 1e-5


def _conv_stats_kernel(x_ref, w1_ref, y_ref, stats_ref, *, H, W, Cin, C1):
    x = x_ref[0]
    acc = jnp.zeros((H * W, C1), jnp.float32)
    for t in range(9):
        ky, kx = t // 3, t % 3
        tap = x[ky:ky + H, kx:kx + W, :].reshape(H * W, Cin)
        acc = acc + jnp.dot(tap, w1_ref[t], preferred_element_type=jnp.float32)
    y_ref[0] = acc
    s = jnp.sum(acc, axis=0, keepdims=True)
    mean = s * (1.0 / (H * W))
    d = acc - mean
    stats_ref[0, 0:1, :] = s
    stats_ref[0, 1:2, :] = jnp.sum(d * d, axis=0, keepdims=True)


def _bn_head_kernel(y_ref, scale_ref, shift_ref, w2_ref, b2_ref, out_ref):
    z = jnp.maximum(y_ref[0] * scale_ref[...] + shift_ref[...], 0.0)
    z = z.astype(jnp.bfloat16)
    out = jax.lax.dot_general(
        w2_ref[...], z, (((0,), (1,)), ((), ())),
        preferred_element_type=jnp.float32)
    out_ref[0] = out + b2_ref[...]


def kernel(x_nchw, w1, b1, gamma, beta, w2, b2):
    del b1
    N, Cin, H, W = x_nchw.shape
    C1 = w1.shape[-1]
    O = w2.shape[-1]
    HW = H * W
    rows = N * HW

    x_pad = jnp.pad(
        jnp.transpose(x_nchw.astype(jnp.bfloat16), (0, 2, 3, 1)),
        ((0, 0), (1, 1), (1, 1), (0, 0)))
    w1b = w1.reshape(9, Cin, C1).astype(jnp.bfloat16)
    w2b = w2.reshape(C1, O).astype(jnp.bfloat16)
    b2c = b2.reshape(O, 1).astype(jnp.float32)

    cparams = pltpu.CompilerParams(
        dimension_semantics=("parallel",),
        vmem_limit_bytes=64 * 1024 * 1024,
    )

    conv_flops = 2 * rows * 9 * Cin * C1
    y, stats = pl.pallas_call(
        functools.partial(_conv_stats_kernel, H=H, W=W, Cin=Cin, C1=C1),
        out_shape=(jax.ShapeDtypeStruct((N, HW, C1), jnp.float32),
                   jax.ShapeDtypeStruct((N, 2, C1), jnp.float32)),
        grid=(N,),
        in_specs=[pl.BlockSpec((1, H + 2, W + 2, Cin), lambda n: (n, 0, 0, 0)),
                  pl.BlockSpec((9, Cin, C1), lambda n: (0, 0, 0))],
        out_specs=(pl.BlockSpec((1, HW, C1), lambda n: (n, 0, 0)),
                   pl.BlockSpec((1, 2, C1), lambda n: (n, 0, 0))),
        compiler_params=cparams,
        cost_estimate=pl.CostEstimate(
            flops=conv_flops, transcendentals=0,
            bytes_accessed=x_pad.size * 2 + w1b.size * 2
            + (rows + 2 * N) * C1 * 4),
    )(x_pad, w1b)

    sums = stats[:, 0, :]
    m2s = stats[:, 1, :]
    mean_i = sums / HW
    mean = jnp.sum(sums, axis=0) / rows
    M2 = jnp.sum(m2s, axis=0) + HW * jnp.sum((mean_i - mean[None, :]) ** 2,
                                             axis=0)
    var = jnp.maximum(M2 / rows, 0.0)
    scale_v = gamma.reshape(C1) * jax.lax.rsqrt(var + _BN_EPS)
    shift_v = beta.reshape(C1) - mean * scale_v
    scale = scale_v[None, :]
    shift = shift_v[None, :]

    out = pl.pallas_call(
        _bn_head_kernel,
        out_shape=jax.ShapeDtypeStruct((N, O, HW), jnp.float32),
        grid=(N,),
        in_specs=[pl.BlockSpec((1, HW, C1), lambda n: (n, 0, 0)),
                  pl.BlockSpec((1, C1), lambda n: (0, 0)),
                  pl.BlockSpec((1, C1), lambda n: (0, 0)),
                  pl.BlockSpec((C1, O), lambda n: (0, 0)),
                  pl.BlockSpec((O, 1), lambda n: (0, 0))],
        out_specs=pl.BlockSpec((1, O, HW), lambda n: (n, 0, 0)),
        compiler_params=cparams,
        cost_estimate=pl.CostEstimate(
            flops=2 * rows * C1 * O, transcendentals=0,
            bytes_accessed=rows * C1 * 4 + w2b.size * 2 + rows * O * 4),
    )(y, scale, shift, w2b, b2c)

    return out.reshape(N, O, H, W)


# speedup vs baseline: 1.1011x; 1.1011x over previous
import functools

import jax
import jax.numpy as jnp
from jax.experimental import pallas as pl
from jax.experimental.pallas import tpu as pltpu

_BN_EPS = 1e-5


def _fused_kernel(x_ref, w1_ref, w2_ref, gb_ref, b2_ref, out_ref,
                  y_ref, xw_ref, st_ref, ss_ref, *, N, H, W, Cin, C1, O):
    g = pl.program_id(0)
    HW = H * W

    @pl.when(g == 0)
    def _init():
        st_ref[...] = jnp.zeros_like(st_ref)

    @pl.when(g < N)
    def _conv_phase():
        x = x_ref[0]
        for kx in range(3):
            xw_ref[:, :, kx * Cin:(kx + 1) * Cin] = (
                x[:, kx:kx + W, :].astype(jnp.bfloat16))
        acc = jnp.zeros((C1, HW), jnp.float32)
        for ky in range(3):
            tap = xw_ref[ky:ky + H].reshape(HW, 3 * Cin)
            acc = acc + jax.lax.dot_general(
                w1_ref[ky], tap, (((0,), (1,)), ((), ())),
                preferred_element_type=jnp.float32)
        y_ref[pl.ds(g, 1)] = acc[None]
        st_ref[:, 0:1] += jnp.sum(acc, axis=1, keepdims=True)
        st_ref[:, 1:2] += jnp.sum(acc * acc, axis=1, keepdims=True)

    @pl.when(g == N)
    def _bn_resolve():
        rows = N * HW
        mean = st_ref[:, 0:1] * (1.0 / rows)
        var = jnp.maximum(st_ref[:, 1:2] * (1.0 / rows) - mean * mean, 0.0)
        scale = gb_ref[:, 0:1] * jax.lax.rsqrt(var + _BN_EPS)
        ss_ref[:, 0:1] = scale
        ss_ref[:, 1:2] = gb_ref[:, 1:2] - mean * scale

    @pl.when(g >= N)
    def _head_phase():
        y = y_ref[pl.ds(g - N, 1)][0]
        z = jnp.maximum(y * ss_ref[:, 0:1] + ss_ref[:, 1:2], 0.0)
        z = z.astype(jnp.bfloat16)
        out = jnp.dot(w2_ref[...], z, preferred_element_type=jnp.float32)
        out_ref[0] = out + b2_ref[...]


def kernel(x_nchw, w1, b1, gamma, beta, w2, b2):
    del b1
    N, Cin, H, W = x_nchw.shape
    C1 = w1.shape[-1]
    O = w2.shape[-1]
    HW = H * W

    x_pad = jnp.pad(
        jnp.transpose(x_nchw, (0, 2, 3, 1)),
        ((0, 0), (1, 1), (1, 1), (0, 0)))
    w1b = w1.reshape(3, 3 * Cin, C1).astype(jnp.bfloat16)
    w2t = w2.reshape(C1, O).T.astype(jnp.bfloat16)
    gb = jnp.stack([gamma.reshape(C1), beta.reshape(C1)], axis=1)
    b2c = b2.reshape(O, 1).astype(jnp.float32)

    out = pl.pallas_call(
        functools.partial(_fused_kernel, N=N, H=H, W=W, Cin=Cin, C1=C1, O=O),
        out_shape=jax.ShapeDtypeStruct((N, O, HW), jnp.float32),
        grid=(2 * N,),
        in_specs=[
            pl.BlockSpec((1, H + 2, W + 2, Cin),
                         lambda g: (jnp.minimum(g, N - 1), 0, 0, 0)),
            pl.BlockSpec((3, 3 * Cin, C1), lambda g: (0, 0, 0)),
            pl.BlockSpec((O, C1), lambda g: (0, 0)),
            pl.BlockSpec((C1, 2), lambda g: (0, 0)),
            pl.BlockSpec((O, 1), lambda g: (0, 0)),
        ],
        out_specs=pl.BlockSpec((1, O, HW),
                               lambda g: (jnp.maximum(g - N, 0), 0, 0)),
        scratch_shapes=[
            pltpu.VMEM((N, C1, HW), jnp.float32),
            pltpu.VMEM((H + 2, W, 3 * Cin), jnp.bfloat16),
            pltpu.VMEM((C1, 2), jnp.float32),
            pltpu.VMEM((C1, 2), jnp.float32),
        ],
        compiler_params=pltpu.CompilerParams(
            dimension_semantics=("arbitrary",),
            vmem_limit_bytes=48 * 1024 * 1024,
        ),
        cost_estimate=pl.CostEstimate(
            flops=2 * N * HW * (9 * Cin + O) * C1, transcendentals=0,
            bytes_accessed=x_pad.size * 4 + N * HW * O * 4
            + (w1b.size + w2t.size) * 2),
    )(x_pad, w1b, w2t, gb, b2c)

    return out.reshape(N, O, H, W)
